# baseline (speedup 1.0000x reference)
import functools

import jax
import jax.numpy as jnp
from jax import lax
from jax.experimental import pallas as pl
from jax.experimental.pallas import tpu as pltpu
from jax.experimental.pallas import tpu_sc as plsc

BOARD_LEN = 10
N_BOARD_CELL = 100
N_OCCUPIABLE_CELL = 92
N_PIECE_ID = 82
N_CH = 96
N_OUT_CH = N_CH + N_PIECE_ID
P_PAD = 104

L = 16


def _cell_of(j):
    j = j.astype(jnp.int32) if hasattr(j, "astype") else jnp.int32(j)
    off = ((j >= 42).astype(jnp.int32) + (j >= 44).astype(jnp.int32)
           + (j >= 48).astype(jnp.int32) + (j >= 50).astype(jnp.int32))
    return j + 2 * off


def _sc_kernel(x_hbm, p_hbm, out_hbm, x_t, p_t, o_t):
    n_workers = 32
    batch = x_hbm.shape[0]
    per_w = batch // n_workers
    wid = lax.axis_index("s") * 2 + lax.axis_index("c")

    lane = jnp.arange(L, dtype=jnp.int32)
    lane100 = lane * 100
    zeros = jnp.zeros((L,), jnp.float32)
    ones = jnp.ones((L,), jnp.float32)

    def zero_row(j, _):
        base = j * N_OUT_CH
        for k in range(6):
            cols = lane + (N_CH + L * k)
            m = cols < N_OUT_CH
            plsc.store_scatter(o_t, [base + jnp.minimum(cols, N_OUT_CH - 1)],
                               zeros, mask=m)
        return 0
    lax.fori_loop(0, N_OCCUPIABLE_CELL, zero_row, 0)

    def onehot_chunks(val):
        for jc in range(6):
            jraw = lane + L * jc
            m = jraw < N_OCCUPIABLE_CELL
            jvec = jnp.minimum(jraw, N_OCCUPIABLE_CELL - 1)
            cellv = _cell_of(jvec)
            pv = plsc.load_gather(p_t, [cellv])
            plsc.store_scatter(o_t, [jvec * N_OUT_CH + (pv + N_CH)], val,
                               mask=m)

    def body(bl, _):
        b = wid * per_w + bl
        pltpu.sync_copy(x_hbm.at[b], x_t)
        pltpu.sync_copy(p_hbm.at[b], p_t)

        def row(j, _):
            cell = _cell_of(j)
            base = j * N_OUT_CH
            for k in range(6):
                idx = lane100 + (k * L * 100 + cell)
                v = plsc.load_gather(x_t, [idx])
                plsc.store_scatter(o_t, [base + lane + L * k], v)
            return 0
        lax.fori_loop(0, N_OCCUPIABLE_CELL, row, 0)

        onehot_chunks(ones)
        pltpu.sync_copy(o_t, out_hbm.at[b])
        onehot_chunks(zeros)
        return 0

    lax.fori_loop(0, per_w, body, 0)


def kernel(infostate_tensor, piece_ids, piece_id_onehot):
    del piece_id_onehot
    B = infostate_tensor.shape[0]
    x = infostate_tensor.reshape(B, N_CH * N_BOARD_CELL)
    p = piece_ids.reshape(B, N_BOARD_CELL).astype(jnp.int32)
    p = jnp.pad(p, ((0, 0), (0, P_PAD - N_BOARD_CELL)))

    run = functools.partial(
        pl.kernel,
        out_type=jax.ShapeDtypeStruct((B, N_OCCUPIABLE_CELL * N_OUT_CH),
                                      jnp.float32),
        mesh=plsc.VectorSubcoreMesh(core_axis_name="c", subcore_axis_name="s"),
        compiler_params=pltpu.CompilerParams(
            use_tc_tiling_on_sc=False, needs_layout_passes=False
        ),
        scratch_types=[
            pltpu.VMEM((N_CH * N_BOARD_CELL,), jnp.float32),
            pltpu.VMEM((P_PAD,), jnp.int32),
            pltpu.VMEM((N_OCCUPIABLE_CELL * N_OUT_CH,), jnp.float32),
        ],
    )(_sc_kernel)
    return run(x, p).reshape(B, N_OCCUPIABLE_CELL, N_OUT_CH)


# speedup vs baseline: 1.1676x; 1.1676x over previous
import functools

import jax
import jax.numpy as jnp
from jax import lax
from jax.experimental import pallas as pl
from jax.experimental.pallas import tpu as pltpu
from jax.experimental.pallas import tpu_sc as plsc

BOARD_LEN = 10
N_BOARD_CELL = 100
N_OCCUPIABLE_CELL = 92
N_PIECE_ID = 82
N_CH = 96
N_OUT_CH = N_CH + N_PIECE_ID
P_PAD = 104
N_WORKERS = 32
L = 16
OUT_WORDS = N_OCCUPIABLE_CELL * N_OUT_CH


def _cell_of(j):
    off = ((j >= 42).astype(jnp.int32) + (j >= 44).astype(jnp.int32)
           + (j >= 48).astype(jnp.int32) + (j >= 50).astype(jnp.int32))
    return j + 2 * off


def _sc_kernel(x_hbm, p_hbm, out_hbm,
               x_t0, x_t1, p_t0, p_t1, o_t0, o_t1, oh_t0, oh_t1,
               sx0, sx1, sp0, sp1, so0, so1):
    batch = x_hbm.shape[0]
    per_w = batch // N_WORKERS
    wid = lax.axis_index("s") * 2 + lax.axis_index("c")
    base_b = wid * per_w

    xs, ps_, os_ = (x_t0, x_t1), (p_t0, p_t1), (o_t0, o_t1)
    ohs, sxs, sps, sos = (oh_t0, oh_t1), (sx0, sx1), (sp0, sp1), (so0, so1)

    lane = jnp.arange(L, dtype=jnp.int32)
    lane100 = lane * 100
    zeros = jnp.zeros((L,), jnp.float32)
    ones = jnp.ones((L,), jnp.float32)
    masks = [lane + L * jc < N_OCCUPIABLE_CELL for jc in range(6)]

    for par in (0, 1):
        o_t = os_[par]

        def zero_row(j, _, o_t=o_t):
            base = j * N_OUT_CH
            for k in range(6):
                cols = lane + (N_CH + L * k)
                m = cols < N_OUT_CH
                plsc.store_scatter(
                    o_t, [base + jnp.minimum(cols, N_OUT_CH - 1)], zeros,
                    mask=m)
            return 0
        lax.fori_loop(0, N_OCCUPIABLE_CELL, zero_row, 0)

    for par in (0, 1):
        pltpu.async_copy(x_hbm.at[base_b + par], xs[par], sxs[par])
        pltpu.async_copy(p_hbm.at[base_b + par], ps_[par], sps[par])

    def body(i, _):
        for par in (0, 1):
            bl = 2 * i + par
            b = base_b + bl
            x_t, p_t, o_t, oh_t = xs[par], ps_[par], os_[par], ohs[par]

            @pl.when(i > 0)
            def _():
                pltpu.make_async_copy(o_t, out_hbm.at[b - 2], sos[par]).wait()
                for jc in range(6):
                    idx = oh_t[pl.ds(jc * L, L)]
                    plsc.store_scatter(o_t, [idx], zeros, mask=masks[jc])

            pltpu.make_async_copy(x_hbm.at[b], x_t, sxs[par]).wait()
            pltpu.make_async_copy(p_hbm.at[b], p_t, sps[par]).wait()

            def row(j, _, x_t=x_t, o_t=o_t):
                cell = _cell_of(j)
                base = j * N_OUT_CH
                for k in range(6):
                    idx = lane100 + (k * L * 100 + cell)
                    v = plsc.load_gather(x_t, [idx])
                    plsc.store_scatter(o_t, [base + lane + L * k], v)
                return 0
            lax.fori_loop(0, N_OCCUPIABLE_CELL, row, 0)

            for jc in range(6):
                jvec = jnp.minimum(lane + L * jc, N_OCCUPIABLE_CELL - 1)
                pv = plsc.load_gather(p_t, [_cell_of(jvec)])
                flat = jvec * N_OUT_CH + (pv + N_CH)
                oh_t[pl.ds(jc * L, L)] = flat
                plsc.store_scatter(o_t, [flat], ones, mask=masks[jc])

            pltpu.async_copy(o_t, out_hbm.at[b], sos[par])

            @pl.when(bl + 2 < per_w)
            def _():
                pltpu.async_copy(x_hbm.at[b + 2], x_t, sxs[par])
                pltpu.async_copy(p_hbm.at[b + 2], p_t, sps[par])
        return 0

    lax.fori_loop(0, per_w // 2, body, 0)

    for par in (0, 1):
        pltpu.make_async_copy(
            os_[par], out_hbm.at[base_b + per_w - 2 + par], sos[par]).wait()


def kernel(infostate_tensor, piece_ids, piece_id_onehot):
    del piece_id_onehot
    B = infostate_tensor.shape[0]
    x = infostate_tensor.reshape(B, N_CH * N_BOARD_CELL)
    p = piece_ids.reshape(B, N_BOARD_CELL).astype(jnp.int32)
    p = jnp.pad(p, ((0, 0), (0, P_PAD - N_BOARD_CELL)))

    run = functools.partial(
        pl.kernel,
        out_type=jax.ShapeDtypeStruct((B, OUT_WORDS), jnp.float32),
        mesh=plsc.VectorSubcoreMesh(core_axis_name="c", subcore_axis_name="s"),
        compiler_params=pltpu.CompilerParams(
            use_tc_tiling_on_sc=False, needs_layout_passes=False
        ),
        scratch_types=[
            pltpu.VMEM((N_CH * N_BOARD_CELL,), jnp.float32),
            pltpu.VMEM((N_CH * N_BOARD_CELL,), jnp.float32),
            pltpu.VMEM((P_PAD,), jnp.int32),
            pltpu.VMEM((P_PAD,), jnp.int32),
            pltpu.VMEM((OUT_WORDS,), jnp.float32),
            pltpu.VMEM((OUT_WORDS,), jnp.float32),
            pltpu.VMEM((6 * L,), jnp.int32),
            pltpu.VMEM((6 * L,), jnp.int32),
            pltpu.SemaphoreType.DMA,
            pltpu.SemaphoreType.DMA,
            pltpu.SemaphoreType.DMA,
            pltpu.SemaphoreType.DMA,
            pltpu.SemaphoreType.DMA,
            pltpu.SemaphoreType.DMA,
        ],
    )(_sc_kernel)
    return run(x, p).reshape(B, N_OCCUPIABLE_CELL, N_OUT_CH)


# speedup vs baseline: 1.3438x; 1.1509x over previous
import functools

import jax
import jax.numpy as jnp
from jax import lax
from jax.experimental import pallas as pl
from jax.experimental.pallas import tpu as pltpu
from jax.experimental.pallas import tpu_sc as plsc

BOARD_LEN = 10
N_BOARD_CELL = 100
N_OCCUPIABLE_CELL = 92
N_PIECE_ID = 82
N_CH = 96
N_OUT_CH = N_CH + N_PIECE_ID
P_PAD = 104
N_WORKERS = 32
L = 16
OUT_WORDS = N_OCCUPIABLE_CELL * N_OUT_CH


def _cell_of(j):
    off = ((j >= 42).astype(jnp.int32) + (j >= 44).astype(jnp.int32)
           + (j >= 48).astype(jnp.int32) + (j >= 50).astype(jnp.int32))
    return j + 2 * off


def _sc_kernel(x_hbm, p_hbm, out_hbm,
               x_t0, x_t1, p_t0, p_t1, o_t0, o_t1, oh_t0, oh_t1,
               sx0, sx1, sp0, sp1, so0, so1):
    batch = x_hbm.shape[0]
    per_w = batch // N_WORKERS
    wid = lax.axis_index("s") * 2 + lax.axis_index("c")
    base_b = wid * per_w

    xs, ps_, os_ = (x_t0, x_t1), (p_t0, p_t1), (o_t0, o_t1)
    ohs, sxs, sps, sos = (oh_t0, oh_t1), (sx0, sx1), (sp0, sp1), (so0, so1)

    lane = jnp.arange(L, dtype=jnp.int32)
    lane100 = lane * 100
    zeros = jnp.zeros((L,), jnp.float32)
    ones = jnp.ones((L,), jnp.float32)
    masks = [lane + L * jc < N_OCCUPIABLE_CELL for jc in range(6)]

    for par in (0, 1):
        o_t = os_[par]

        def zero_row(j, _, o_t=o_t):
            rows = jnp.full((L,), j, jnp.int32)
            for k in range(6):
                cols = lane + (N_CH + L * k)
                m = cols < N_OUT_CH
                plsc.store_scatter(
                    o_t, [rows, jnp.minimum(cols, N_OUT_CH - 1)], zeros,
                    mask=m)
            return 0
        lax.fori_loop(0, N_OCCUPIABLE_CELL, zero_row, 0)

    for par in (0, 1):
        pltpu.async_copy(x_hbm.at[base_b + par], xs[par], sxs[par])
        pltpu.async_copy(p_hbm.at[base_b + par], ps_[par], sps[par])

    def body(i, _):
        for par in (0, 1):
            bl = 2 * i + par
            b = base_b + bl
            x_t, p_t, o_t, oh_t = xs[par], ps_[par], os_[par], ohs[par]

            @pl.when(i > 0)
            def _():
                pltpu.make_async_copy(o_t, out_hbm.at[b - 2], sos[par]).wait()
                for jc in range(6):
                    rows = oh_t[pl.ds(jc * L, L)]
                    cols = oh_t[pl.ds(96 + jc * L, L)]
                    plsc.store_scatter(o_t, [rows, cols], zeros,
                                       mask=masks[jc])

            pltpu.make_async_copy(x_hbm.at[b], x_t, sxs[par]).wait()
            pltpu.make_async_copy(p_hbm.at[b], p_t, sps[par]).wait()

            def row(j, _, x_t=x_t, o_t=o_t):
                cell = _cell_of(j)
                rows = jnp.full((L,), j, jnp.int32)
                for k in range(6):
                    idx = lane100 + (k * L * 100 + cell)
                    v = plsc.load_gather(x_t, [idx])
                    plsc.store_scatter(o_t, [rows, lane + L * k], v)
                return 0
            lax.fori_loop(0, N_OCCUPIABLE_CELL, row, 0)

            for jc in range(6):
                jvec = jnp.minimum(lane + L * jc, N_OCCUPIABLE_CELL - 1)
                pv = plsc.load_gather(p_t, [_cell_of(jvec)])
                cols = pv + N_CH
                oh_t[pl.ds(jc * L, L)] = jvec
                oh_t[pl.ds(96 + jc * L, L)] = cols
                plsc.store_scatter(o_t, [jvec, cols], ones, mask=masks[jc])

            pltpu.async_copy(o_t, out_hbm.at[b], sos[par])

            @pl.when(bl + 2 < per_w)
            def _():
                pltpu.async_copy(x_hbm.at[b + 2], x_t, sxs[par])
                pltpu.async_copy(p_hbm.at[b + 2], p_t, sps[par])
        return 0

    lax.fori_loop(0, per_w // 2, body, 0)

    for par in (0, 1):
        pltpu.make_async_copy(
            os_[par], out_hbm.at[base_b + per_w - 2 + par], sos[par]).wait()


def kernel(infostate_tensor, piece_ids, piece_id_onehot):
    del piece_id_onehot
    B = infostate_tensor.shape[0]
    x = infostate_tensor.reshape(B, N_CH * N_BOARD_CELL)
    p = piece_ids.reshape(B, N_BOARD_CELL).astype(jnp.int32)
    p = jnp.pad(p, ((0, 0), (0, P_PAD - N_BOARD_CELL)))

    run = functools.partial(
        pl.kernel,
        out_type=jax.ShapeDtypeStruct(
            (B, N_OCCUPIABLE_CELL, N_OUT_CH), jnp.float32),
        mesh=plsc.VectorSubcoreMesh(core_axis_name="c", subcore_axis_name="s"),
        compiler_params=pltpu.CompilerParams(
            use_tc_tiling_on_sc=False, needs_layout_passes=False
        ),
        scratch_types=[
            pltpu.VMEM((N_CH * N_BOARD_CELL,), jnp.float32),
            pltpu.VMEM((N_CH * N_BOARD_CELL,), jnp.float32),
            pltpu.VMEM((P_PAD,), jnp.int32),
            pltpu.VMEM((P_PAD,), jnp.int32),
            pltpu.VMEM((N_OCCUPIABLE_CELL, N_OUT_CH), jnp.float32),
            pltpu.VMEM((N_OCCUPIABLE_CELL, N_OUT_CH), jnp.float32),
            pltpu.VMEM((2 * 6 * L,), jnp.int32),
            pltpu.VMEM((2 * 6 * L,), jnp.int32),
            pltpu.SemaphoreType.DMA,
            pltpu.SemaphoreType.DMA,
            pltpu.SemaphoreType.DMA,
            pltpu.SemaphoreType.DMA,
            pltpu.SemaphoreType.DMA,
            pltpu.SemaphoreType.DMA,
        ],
    )(_sc_kernel)
    return run(x, p)


# speedup vs baseline: 1.7908x; 1.3327x over previous
import functools

import jax
import jax.numpy as jnp
from jax import lax
from jax.experimental import pallas as pl
from jax.experimental.pallas import tpu as pltpu
from jax.experimental.pallas import tpu_sc as plsc

BOARD_LEN = 10
N_BOARD_CELL = 100
N_OCCUPIABLE_CELL = 92
N_PIECE_ID = 82
N_CH = 96
N_OUT_CH = N_CH + N_PIECE_ID
P_PAD = 104
N_WORKERS = 32
L = 16
OUT_WORDS = N_OCCUPIABLE_CELL * N_OUT_CH


def _cell_of(j):
    off = ((j >= 42).astype(jnp.int32) + (j >= 44).astype(jnp.int32)
           + (j >= 48).astype(jnp.int32) + (j >= 50).astype(jnp.int32))
    return j + 2 * off


def _sc_kernel(x_hbm, p_hbm, out_hbm,
               x_t0, x_t1, p_t0, p_t1, o_t0, o_t1, oh_t0, oh_t1,
               sx0, sx1, sp0, sp1, so0, so1):
    batch = x_hbm.shape[0]
    per_w = batch // N_WORKERS
    wid = lax.axis_index("s") * 2 + lax.axis_index("c")
    base_b = wid * per_w

    xs, ps_, os_ = (x_t0, x_t1), (p_t0, p_t1), (o_t0, o_t1)
    ohs, sxs, sps, sos = (oh_t0, oh_t1), (sx0, sx1), (sp0, sp1), (so0, so1)

    lane = jnp.arange(L, dtype=jnp.int32)
    lane100 = lane * 100
    zeros = jnp.zeros((L,), jnp.float32)
    ones = jnp.ones((L,), jnp.float32)
    masks = [lane + L * jc < N_OCCUPIABLE_CELL for jc in range(6)]

    for par in (0, 1):
        o_t = os_[par]

        def zero_row(j, _, o_t=o_t):
            rows = jnp.full((L,), j, jnp.int32)
            for k in range(6):
                cols = lane + (N_CH + L * k)
                m = cols < N_OUT_CH
                plsc.store_scatter(
                    o_t, [rows, jnp.minimum(cols, N_OUT_CH - 1)], zeros,
                    mask=m)
            return 0
        lax.fori_loop(0, N_OCCUPIABLE_CELL, zero_row, 0)

    for par in (0, 1):
        pltpu.async_copy(x_hbm.at[base_b + par], xs[par], sxs[par])
        pltpu.async_copy(p_hbm.at[base_b + par], ps_[par], sps[par])

    def body(i, _):
        for par in (0, 1):
            bl = 2 * i + par
            b = base_b + bl
            x_t, p_t, o_t, oh_t = xs[par], ps_[par], os_[par], ohs[par]

            @pl.when(i > 0)
            def _():
                pltpu.make_async_copy(o_t, out_hbm.at[b - 2], sos[par]).wait()
                for jc in range(6):
                    rows = oh_t[pl.ds(jc * L, L)]
                    cols = oh_t[pl.ds(96 + jc * L, L)]
                    plsc.store_scatter(o_t, [rows, cols], zeros,
                                       mask=masks[jc])

            pltpu.make_async_copy(x_hbm.at[b], x_t, sxs[par]).wait()
            pltpu.make_async_copy(p_hbm.at[b], p_t, sps[par]).wait()

            def row(j, _, x_t=x_t, o_t=o_t):
                cell = _cell_of(j)
                rows = jnp.full((L,), j, jnp.int32)
                for k in range(6):
                    idx = lane100 + (k * L * 100 + cell)
                    v = plsc.load_gather(x_t, [idx])
                    plsc.store_scatter(o_t, [rows, lane + L * k], v)
                return 0
            lax.fori_loop(0, N_OCCUPIABLE_CELL, row, 0)

            for jc in range(6):
                jvec = jnp.minimum(lane + L * jc, N_OCCUPIABLE_CELL - 1)
                pv = plsc.load_gather(p_t, [_cell_of(jvec)])
                cols = pv + N_CH
                oh_t[pl.ds(jc * L, L)] = jvec
                oh_t[pl.ds(96 + jc * L, L)] = cols
                plsc.store_scatter(o_t, [jvec, cols], ones, mask=masks[jc])

            pltpu.async_copy(o_t, out_hbm.at[b], sos[par])

            @pl.when(bl + 2 < per_w)
            def _():
                pltpu.async_copy(x_hbm.at[b + 2], x_t, sxs[par])
                pltpu.async_copy(p_hbm.at[b + 2], p_t, sps[par])
        return 0

    lax.fori_loop(0, per_w // 2, body, 0)

    for par in (0, 1):
        pltpu.make_async_copy(
            os_[par], out_hbm.at[base_b + per_w - 2 + par], sos[par]).wait()


def kernel(infostate_tensor, piece_ids, piece_id_onehot):
    del piece_id_onehot
    B = infostate_tensor.shape[0]
    x = infostate_tensor.reshape(B, N_CH * N_BOARD_CELL)
    p = piece_ids.reshape(B, N_BOARD_CELL).astype(jnp.int32)
    p = jnp.pad(p, ((0, 0), (0, P_PAD - N_BOARD_CELL)))

    run = functools.partial(
        pl.kernel,
        out_type=jax.ShapeDtypeStruct(
            (B, N_OCCUPIABLE_CELL, N_OUT_CH), jnp.float32),
        mesh=plsc.VectorSubcoreMesh(core_axis_name="c", subcore_axis_name="s"),
        compiler_params=pltpu.CompilerParams(
            use_tc_tiling_on_sc=True, needs_layout_passes=False
        ),
        scratch_types=[
            pltpu.VMEM((N_CH * N_BOARD_CELL,), jnp.float32),
            pltpu.VMEM((N_CH * N_BOARD_CELL,), jnp.float32),
            pltpu.VMEM((P_PAD,), jnp.int32),
            pltpu.VMEM((P_PAD,), jnp.int32),
            pltpu.VMEM((N_OCCUPIABLE_CELL, N_OUT_CH), jnp.float32),
            pltpu.VMEM((N_OCCUPIABLE_CELL, N_OUT_CH), jnp.float32),
            pltpu.VMEM((2 * 6 * L,), jnp.int32),
            pltpu.VMEM((2 * 6 * L,), jnp.int32),
            pltpu.SemaphoreType.DMA,
            pltpu.SemaphoreType.DMA,
            pltpu.SemaphoreType.DMA,
            pltpu.SemaphoreType.DMA,
            pltpu.SemaphoreType.DMA,
            pltpu.SemaphoreType.DMA,
        ],
    )(_sc_kernel)
    return run(x, p)
